# baseline (speedup 1.0000x reference)
import functools

import jax
import jax.numpy as jnp
from jax import lax
from jax.experimental import pallas as pl
from jax.experimental.pallas import tpu as pltpu
from jax.experimental.pallas import tpu_sc as plsc

N = 50000
E = 800000
EMB = 32
HID = 64
IN_DIM = 64

NC = 2
NS = 16
LANES = 16

NPAD = 50176
ROWS_PER_TILE = NPAD // NS
EPAD = 819200
CHUNK = 512
JROWS = CHUNK // 128
EDGE_ROWS = EPAD // 128

BN = 512
GRID = NPAD // BN


def _mesh():
    return plsc.VectorSubcoreMesh(
        core_axis_name="c", subcore_axis_name="s", num_cores=NC, num_subcores=NS
    )


_SC_PARAMS = pltpu.CompilerParams(use_tc_tiling_on_sc=False)


def _splat(v16, t):
    idx = jnp.full((LANES,), t, jnp.int32)
    return lax.gather(
        v16, idx[:, None],
        lax.GatherDimensionNumbers(offset_dims=(), collapsed_slice_dims=(0,),
                                   start_index_map=(0,)),
        (1,), mode=lax.GatherScatterMode.PROMISE_IN_BOUNDS)


def _deg_body(dst2_hbm, w2_hbm, out_hbm, acc, idx_v, w_v, zbuf):
    c = lax.axis_index("c")
    s = lax.axis_index("s")
    r0 = s * ROWS_PER_TILE

    @pl.loop(0, ROWS_PER_TILE // LANES)
    def _zero(k):
        zbuf[pl.ds(k * LANES, LANES)] = jnp.zeros((LANES,), jnp.float32)

    pltpu.sync_copy(zbuf, acc.at[pl.ds(r0, ROWS_PER_TILE)])
    plsc.subcore_barrier()

    nchunks = EPAD // 2 // NS // CHUNK
    base = c * (EDGE_ROWS // 2) + s * (nchunks * JROWS)

    @pl.loop(0, nchunks)
    def _chunk(g):
        row = base + g * JROWS
        pltpu.sync_copy(dst2_hbm.at[pl.ds(row, JROWS)], idx_v)
        pltpu.sync_copy(w2_hbm.at[pl.ds(row, JROWS)], w_v)
        for j in range(JROWS):
            pltpu.sync_copy(w_v.at[j], acc.at[idx_v.at[j]], add=True)

    plsc.subcore_barrier()
    pltpu.sync_copy(acc.at[pl.ds(r0, ROWS_PER_TILE)], zbuf)
    pltpu.sync_copy(zbuf, out_hbm.at[pl.ds(c * NPAD + r0, ROWS_PER_TILE)])


def _deg_call(dst2, w2):
    return pl.kernel(
        _deg_body,
        out_type=jax.ShapeDtypeStruct((NC * NPAD,), jnp.float32),
        mesh=_mesh(),
        scratch_types=[
            pltpu.VMEM_SHARED((NPAD,), jnp.float32),
            pltpu.VMEM((JROWS, 128), jnp.int32),
            pltpu.VMEM((JROWS, 128), jnp.float32),
            pltpu.VMEM((ROWS_PER_TILE,), jnp.float32),
        ],
        compiler_params=_SC_PARAMS,
        name="sc_degree",
    )(dst2, w2)


def _agg_body(y2_hbm, src2_hbm, dst2_hbm, w2_hbm, out_hbm,
              acc, idx_v, idx2_v, dst_v, w_v, rows_v,
              sem_i, sem_g, sem_s, *, hc, ck):
    c = lax.axis_index("c")
    s = lax.axis_index("s")
    r0 = s * ROWS_PER_TILE
    cb = ck // 128
    nchunks = EPAD // NS // ck

    @pl.loop(0, ck)
    def _zero(r):
        for h in range(hc // LANES):
            rows_v[0][r, pl.ds(h * LANES, LANES)] = jnp.zeros(
                (LANES,), jnp.float32)

    nfull = ROWS_PER_TILE // ck
    rem = ROWS_PER_TILE % ck
    for q in range(nfull):
        pltpu.sync_copy(rows_v[0], acc.at[pl.ds(r0 + q * ck, ck)])
    if rem:
        pltpu.sync_copy(rows_v[0].at[pl.ds(0, rem)],
                        acc.at[pl.ds(r0 + nfull * ck, rem)])
    plsc.subcore_barrier()

    def fire_idx(g, m):
        row = s * (nchunks * cb) + g * cb
        pltpu.async_copy(src2_hbm.at[pl.ds(row, cb)], idx_v[m], sem_i[m])
        pltpu.async_copy(dst2_hbm.at[pl.ds(row, cb)], dst_v[m], sem_i[m])
        pltpu.async_copy(w2_hbm.at[pl.ds(row, cb)], w_v[m], sem_i[m])

    def wait_idx(g, m):
        row = s * (nchunks * cb) + g * cb
        pltpu.make_async_copy(src2_hbm.at[pl.ds(row, cb)], idx_v[m],
                              sem_i[m]).wait()
        pltpu.make_async_copy(dst2_hbm.at[pl.ds(row, cb)], dst_v[m],
                              sem_i[m]).wait()
        pltpu.make_async_copy(w2_hbm.at[pl.ds(row, cb)], w_v[m],
                              sem_i[m]).wait()

    def mkidx2(m):
        @pl.loop(0, cb * 8)
        def _mkidx(k):
            j, k16 = k // 8, (k % 8) * LANES
            idx2_v[m][j, pl.ds(k16, LANES)] = (
                idx_v[m][j, pl.ds(k16, LANES)] * 2 + c)

    def fire_gather(p, m):
        for j in range(cb):
            pltpu.async_copy(y2_hbm.at[idx2_v[m].at[j]],
                             rows_v[p].at[pl.ds(j * 128, 128)], sem_g[p])

    def wait_gather(p, m):
        for j in range(cb):
            pltpu.make_async_copy(y2_hbm.at[idx2_v[m].at[j]],
                                  rows_v[p].at[pl.ds(j * 128, 128)],
                                  sem_g[p]).wait()

    def fire_scatter(p, m):
        for j in range(cb):
            pltpu.async_copy(rows_v[p].at[pl.ds(j * 128, 128)],
                             acc.at[dst_v[m].at[j]], sem_s[p], add=True)

    def wait_scatter(p, m):
        for j in range(cb):
            pltpu.make_async_copy(rows_v[p].at[pl.ds(j * 128, 128)],
                                  acc.at[dst_v[m].at[j]], sem_s[p]).wait()

    def scale(p, m):
        @pl.loop(0, ck // LANES)
        def _scale(k):
            j = k // 8
            w16 = w_v[m][j, pl.ds((k % 8) * LANES, LANES)]
            for t in range(LANES):
                r = k * LANES + t
                wk = _splat(w16, t)
                for h in range(hc // LANES):
                    sl = pl.ds(h * LANES, LANES)
                    rows_v[p][r, sl] = rows_v[p][r, sl] * wk

    def stage(g, u, first=False):
        wait_gather(u, u)
        scale(u, u)
        fire_scatter(u, u)
        if not first:
            wait_scatter((u - 1) % 4, (u - 1) % 4)
        m2 = (u + 2) % 4
        wait_idx(g + 2, m2)
        mkidx2(m2)
        fire_gather(m2, m2)
        fire_idx(g + 3, (u + 3) % 4)

    fire_idx(0, 0)
    fire_idx(1, 1)
    fire_idx(2, 2)
    wait_idx(0, 0)
    mkidx2(0)
    fire_gather(0, 0)
    wait_idx(1, 1)
    mkidx2(1)
    fire_gather(1, 1)
    stage(0, 0, first=True)
    stage(1, 1)
    stage(2, 2)
    stage(3, 3)

    @pl.loop(1, nchunks // 4)
    def _pipe(i):
        for u in range(4):
            stage(4 * i + u, u)

    wait_gather(nchunks % 4, nchunks % 4)
    wait_gather((nchunks + 1) % 4, (nchunks + 1) % 4)
    wait_idx(nchunks + 2, (nchunks + 2) % 4)
    wait_scatter((nchunks - 1) % 4, (nchunks - 1) % 4)

    plsc.subcore_barrier()
    for q in range(nfull):
        pltpu.sync_copy(acc.at[pl.ds(r0 + q * ck, ck)], rows_v[0])
        pltpu.sync_copy(rows_v[0], out_hbm.at[c, pl.ds(r0 + q * ck, ck)])
    if rem:
        pltpu.sync_copy(acc.at[pl.ds(r0 + nfull * ck, rem)],
                        rows_v[0].at[pl.ds(0, rem)])
        pltpu.sync_copy(rows_v[0].at[pl.ds(0, rem)],
                        out_hbm.at[c, pl.ds(r0 + nfull * ck, rem)])


def _agg_call(y2, src2, dst2, w2, *, hc):
    ck = 128 if hc == 32 else 256
    cb = ck // 128
    body = functools.partial(_agg_body, hc=hc, ck=ck)
    return pl.kernel(
        body,
        out_type=jax.ShapeDtypeStruct((NC, NPAD, hc), jnp.float32),
        mesh=_mesh(),
        scratch_types=[
            pltpu.VMEM_SHARED((NPAD, hc), jnp.float32),
            [pltpu.VMEM((cb, 128), jnp.int32) for _ in range(4)],
            [pltpu.VMEM((cb, 128), jnp.int32) for _ in range(4)],
            [pltpu.VMEM((cb, 128), jnp.int32) for _ in range(4)],
            [pltpu.VMEM((cb, 128), jnp.float32) for _ in range(4)],
            [pltpu.VMEM((ck, hc), jnp.float32) for _ in range(4)],
            [pltpu.SemaphoreType.DMA for _ in range(4)],
            [pltpu.SemaphoreType.DMA for _ in range(4)],
            [pltpu.SemaphoreType.DMA for _ in range(4)],
        ],
        compiler_params=_SC_PARAMS,
        name=f"sc_agg{hc}",
    )(y2, src2, dst2, w2)


def _l1_body(emb_ref, w_ref, d0_ref, d1_ref, y_ref, dis_ref):
    deg = d0_ref[...] + d1_ref[...] + 1.0
    dis = jnp.where(deg > 0, lax.rsqrt(deg), 0.0)
    dis_ref[...] = dis
    y_ref[...] = dis * jnp.dot(emb_ref[...], w_ref[...],
                               preferred_element_type=jnp.float32)


def _l1_call(emb_p, W1e, deg0, deg1):
    return pl.pallas_call(
        _l1_body,
        grid=(GRID,),
        in_specs=[
            pl.BlockSpec((BN, EMB), lambda i: (i, 0)),
            pl.BlockSpec((EMB, HID), lambda i: (0, 0)),
            pl.BlockSpec((BN, 1), lambda i: (i, 0)),
            pl.BlockSpec((BN, 1), lambda i: (i, 0)),
        ],
        out_specs=[
            pl.BlockSpec((BN, HID), lambda i: (i, 0)),
            pl.BlockSpec((BN, 1), lambda i: (i, 0)),
        ],
        out_shape=[
            jax.ShapeDtypeStruct((NPAD, HID), jnp.float32),
            jax.ShapeDtypeStruct((NPAD, 1), jnp.float32),
        ],
        name="tc_layer1",
    )(emb_p, W1e, deg0, deg1)


def _mid_body(sl_ref, sr_ref, y_ref, dis_ref, b_ref, w_ref, out_ref, x_ref):
    dis = dis_ref[...]
    agg = jnp.concatenate([sl_ref[...], sr_ref[...]], axis=1) + y_ref[...]
    x = jnp.maximum(dis * agg + b_ref[...], 0.0)
    if x_ref is not None:
        x_ref[...] = x
    out_ref[...] = dis * jnp.dot(x, w_ref[...],
                                 preferred_element_type=jnp.float32)


def _mid_call(sL, sR, y, dis, b, W, *, with_x):
    hcin = sL.shape[-1]
    din = 2 * hcin
    dout = W.shape[1]
    if with_x:
        body = _mid_body
        out_specs = [pl.BlockSpec((BN, dout), lambda i: (i, 0)),
                     pl.BlockSpec((BN, din), lambda i: (i, 0))]
        out_shape = [jax.ShapeDtypeStruct((NPAD, dout), jnp.float32),
                     jax.ShapeDtypeStruct((NPAD, din), jnp.float32)]
    else:
        body = functools.partial(_mid_body, x_ref=None)
        out_specs = [pl.BlockSpec((BN, dout), lambda i: (i, 0))]
        out_shape = [jax.ShapeDtypeStruct((NPAD, dout), jnp.float32)]
    res = pl.pallas_call(
        body,
        grid=(GRID,),
        in_specs=[
            pl.BlockSpec((BN, hcin), lambda i: (i, 0)),
            pl.BlockSpec((BN, hcin), lambda i: (i, 0)),
            pl.BlockSpec((BN, din), lambda i: (i, 0)),
            pl.BlockSpec((BN, 1), lambda i: (i, 0)),
            pl.BlockSpec((1, din), lambda i: (0, 0)),
            pl.BlockSpec((din, dout), lambda i: (0, 0)),
        ],
        out_specs=out_specs,
        out_shape=out_shape,
        name=f"tc_layer_{din}_{dout}",
    )(sL, sR, y, dis, b, W)
    return res if with_x else res[0]


def _fin_body(sl_ref, sr_ref, y_ref, dis_ref, b_ref, out_ref):
    agg = jnp.concatenate([sl_ref[...], sr_ref[...]], axis=1) + y_ref[...]
    out_ref[...] = dis_ref[...] * agg + b_ref[...]


def _fin_call(sL, sR, y, dis, b):
    hcin = sL.shape[-1]
    din = 2 * hcin
    return pl.pallas_call(
        _fin_body,
        grid=(GRID,),
        in_specs=[
            pl.BlockSpec((BN, hcin), lambda i: (i, 0)),
            pl.BlockSpec((BN, hcin), lambda i: (i, 0)),
            pl.BlockSpec((BN, din), lambda i: (i, 0)),
            pl.BlockSpec((BN, 1), lambda i: (i, 0)),
            pl.BlockSpec((1, din), lambda i: (0, 0)),
        ],
        out_specs=pl.BlockSpec((BN, din), lambda i: (i, 0)),
        out_shape=jax.ShapeDtypeStruct((NPAD, din), jnp.float32),
        name="tc_final",
    )(sL, sR, y, dis, b)


def kernel(edge_index, edge_weight, embedding, W1e, b1e, W2e, b2e,
           W1d, b1d, W2d, b2d):
    src = edge_index[0]
    dst = edge_index[1]
    pad = EPAD + 16 * 128 - E
    src2 = jnp.concatenate([src, jnp.zeros((pad,), src.dtype)]).reshape(
        EDGE_ROWS + 16, 128)
    dst2 = jnp.concatenate([dst, jnp.zeros((pad,), dst.dtype)]).reshape(
        EDGE_ROWS + 16, 128)
    w2 = jnp.concatenate([edge_weight,
                          jnp.zeros((pad,), edge_weight.dtype)]).reshape(
        EDGE_ROWS + 16, 128)
    emb_p = jnp.pad(embedding, ((0, NPAD - N), (0, 0)))

    deg2 = _deg_call(dst2, w2).reshape(NC, NPAD)
    deg0 = deg2[0].reshape(NPAD, 1)
    deg1 = deg2[1].reshape(NPAD, 1)

    y1, dis = _l1_call(emb_p, W1e, deg0, deg1)
    S1 = _agg_call(y1.reshape(2 * NPAD, HID // 2), src2, dst2, w2, hc=HID // 2)

    y2 = _mid_call(S1[0], S1[1], y1, dis, b1e.reshape(1, HID), W2e,
                   with_x=False)
    S2 = _agg_call(y2.reshape(2 * NPAD, EMB // 2), src2, dst2, w2, hc=EMB // 2)

    y3, x3 = _mid_call(S2[0], S2[1], y2, dis, b2e.reshape(1, EMB), W1d,
                       with_x=True)
    S3 = _agg_call(y3.reshape(2 * NPAD, HID // 2), src2, dst2, w2, hc=HID // 2)

    y4 = _mid_call(S3[0], S3[1], y3, dis, b1d.reshape(1, HID), W2d,
                   with_x=False)
    S4 = _agg_call(y4.reshape(2 * NPAD, IN_DIM // 2), src2, dst2, w2,
                   hc=IN_DIM // 2)

    recon = _fin_call(S4[0], S4[1], y4, dis, b2d.reshape(1, IN_DIM))
    return recon[:N], x3[:N]


# speedup vs baseline: 1.2223x; 1.2223x over previous
import functools

import jax
import jax.numpy as jnp
from jax import lax
from jax.experimental import pallas as pl
from jax.experimental.pallas import tpu as pltpu
from jax.experimental.pallas import tpu_sc as plsc

N = 50000
E = 800000
EMB = 32
HID = 64
IN_DIM = 64

NC = 2
NS = 16
LANES = 16

NPAD = 50176
ROWS_PER_TILE = NPAD // NS
EPAD = 819200
CHUNK = 512
JROWS = CHUNK // 128
EDGE_ROWS = EPAD // 128

BN = 512
GRID = NPAD // BN


def _mesh():
    return plsc.VectorSubcoreMesh(
        core_axis_name="c", subcore_axis_name="s", num_cores=NC, num_subcores=NS
    )


_SC_PARAMS = pltpu.CompilerParams(use_tc_tiling_on_sc=False)


def _splat(v16, t):
    idx = jnp.full((LANES,), t, jnp.int32)
    return lax.gather(
        v16, idx[:, None],
        lax.GatherDimensionNumbers(offset_dims=(), collapsed_slice_dims=(0,),
                                   start_index_map=(0,)),
        (1,), mode=lax.GatherScatterMode.PROMISE_IN_BOUNDS)


def _deg_body(dst2_hbm, w2_hbm, out_hbm, acc, idx_v, w_v, zbuf):
    c = lax.axis_index("c")
    s = lax.axis_index("s")
    r0 = s * ROWS_PER_TILE

    @pl.loop(0, ROWS_PER_TILE // LANES)
    def _zero(k):
        zbuf[pl.ds(k * LANES, LANES)] = jnp.zeros((LANES,), jnp.float32)

    pltpu.sync_copy(zbuf, acc.at[pl.ds(r0, ROWS_PER_TILE)])
    plsc.subcore_barrier()

    nchunks = EPAD // 2 // NS // CHUNK
    base = c * (EDGE_ROWS // 2) + s * (nchunks * JROWS)

    @pl.loop(0, nchunks)
    def _chunk(g):
        row = base + g * JROWS
        pltpu.sync_copy(dst2_hbm.at[pl.ds(row, JROWS)], idx_v)
        pltpu.sync_copy(w2_hbm.at[pl.ds(row, JROWS)], w_v)
        for j in range(JROWS):
            pltpu.sync_copy(w_v.at[j], acc.at[idx_v.at[j]], add=True)

    plsc.subcore_barrier()
    pltpu.sync_copy(acc.at[pl.ds(r0, ROWS_PER_TILE)], zbuf)
    pltpu.sync_copy(zbuf, out_hbm.at[pl.ds(c * NPAD + r0, ROWS_PER_TILE)])


def _deg_call(dst2, w2):
    return pl.kernel(
        _deg_body,
        out_type=jax.ShapeDtypeStruct((NC * NPAD,), jnp.float32),
        mesh=_mesh(),
        scratch_types=[
            pltpu.VMEM_SHARED((NPAD,), jnp.float32),
            pltpu.VMEM((JROWS, 128), jnp.int32),
            pltpu.VMEM((JROWS, 128), jnp.float32),
            pltpu.VMEM((ROWS_PER_TILE,), jnp.float32),
        ],
        compiler_params=_SC_PARAMS,
        name="sc_degree",
    )(dst2, w2)


def _agg_body(y2_hbm, src2_hbm, dst2_hbm, w2_hbm, out_hbm,
              acc, idx_v, idx2_v, dst_v, w_v, rows_v,
              sem_i, sem_g, sem_s, *, hc, ck):
    c = lax.axis_index("c")
    s = lax.axis_index("s")
    r0 = s * ROWS_PER_TILE
    cb = ck // 128
    nchunks = EPAD // NS // ck

    @pl.loop(0, ck)
    def _zero(r):
        for h in range(hc // LANES):
            rows_v[0][r, pl.ds(h * LANES, LANES)] = jnp.zeros(
                (LANES,), jnp.float32)

    nfull = ROWS_PER_TILE // ck
    rem = ROWS_PER_TILE % ck
    for q in range(nfull):
        pltpu.sync_copy(rows_v[0], acc.at[pl.ds(r0 + q * ck, ck)])
    if rem:
        pltpu.sync_copy(rows_v[0].at[pl.ds(0, rem)],
                        acc.at[pl.ds(r0 + nfull * ck, rem)])
    plsc.subcore_barrier()

    def fire_idx(g, m):
        row = s * (nchunks * cb) + g * cb
        pltpu.async_copy(src2_hbm.at[pl.ds(row, cb)], idx_v[m], sem_i[m])
        pltpu.async_copy(dst2_hbm.at[pl.ds(row, cb)], dst_v[m], sem_i[m])
        pltpu.async_copy(w2_hbm.at[pl.ds(row, cb)], w_v[m], sem_i[m])

    def wait_idx(g, m):
        row = s * (nchunks * cb) + g * cb
        pltpu.make_async_copy(src2_hbm.at[pl.ds(row, cb)], idx_v[m],
                              sem_i[m]).wait()
        pltpu.make_async_copy(dst2_hbm.at[pl.ds(row, cb)], dst_v[m],
                              sem_i[m]).wait()
        pltpu.make_async_copy(w2_hbm.at[pl.ds(row, cb)], w_v[m],
                              sem_i[m]).wait()

    def mkidx2(m):
        @pl.loop(0, cb * 8)
        def _mkidx(k):
            j, k16 = k // 8, (k % 8) * LANES
            idx2_v[m][j, pl.ds(k16, LANES)] = (
                idx_v[m][j, pl.ds(k16, LANES)] * 2 + c)

    def fire_gather(p, m):
        for j in range(cb):
            pltpu.async_copy(y2_hbm.at[idx2_v[m].at[j]],
                             rows_v[p].at[pl.ds(j * 128, 128)], sem_g[p])

    def wait_gather(p, m):
        for j in range(cb):
            pltpu.make_async_copy(y2_hbm.at[idx2_v[m].at[j]],
                                  rows_v[p].at[pl.ds(j * 128, 128)],
                                  sem_g[p]).wait()

    def fire_scatter(p, m):
        for j in range(cb):
            pltpu.async_copy(rows_v[p].at[pl.ds(j * 128, 128)],
                             acc.at[dst_v[m].at[j]], sem_s[p], add=True)

    def wait_scatter(p, m):
        for j in range(cb):
            pltpu.make_async_copy(rows_v[p].at[pl.ds(j * 128, 128)],
                                  acc.at[dst_v[m].at[j]], sem_s[p]).wait()

    def scale(p, m):
        @pl.loop(0, ck // LANES)
        def _scale(k):
            j = k // 8
            w16 = w_v[m][j, pl.ds((k % 8) * LANES, LANES)]
            for t in range(LANES):
                r = k * LANES + t
                wk = _splat(w16, t)
                for h in range(hc // LANES):
                    sl = pl.ds(h * LANES, LANES)
                    rows_v[p][r, sl] = rows_v[p][r, sl] * wk

    def stage(g, u, first=False):
        wait_gather(u, u)
        scale(u, u)
        fire_scatter(u, u)
        if not first:
            wait_scatter((u - 1) % 4, (u - 1) % 4)
        m2 = (u + 2) % 4
        wait_idx(g + 2, m2)
        mkidx2(m2)
        fire_gather(m2, m2)
        fire_idx(g + 3, (u + 3) % 4)

    fire_idx(0, 0)
    fire_idx(1, 1)
    fire_idx(2, 2)
    wait_idx(0, 0)
    mkidx2(0)
    fire_gather(0, 0)
    wait_idx(1, 1)
    mkidx2(1)
    fire_gather(1, 1)
    stage(0, 0, first=True)
    stage(1, 1)
    stage(2, 2)
    stage(3, 3)

    @pl.loop(1, nchunks // 4)
    def _pipe(i):
        for u in range(4):
            stage(4 * i + u, u)

    wait_gather(nchunks % 4, nchunks % 4)
    wait_gather((nchunks + 1) % 4, (nchunks + 1) % 4)
    wait_idx(nchunks + 2, (nchunks + 2) % 4)
    wait_scatter((nchunks - 1) % 4, (nchunks - 1) % 4)

    plsc.subcore_barrier()
    for q in range(nfull):
        pltpu.sync_copy(acc.at[pl.ds(r0 + q * ck, ck)], rows_v[0])
        pltpu.sync_copy(rows_v[0], out_hbm.at[c, pl.ds(r0 + q * ck, ck)])
    if rem:
        pltpu.sync_copy(acc.at[pl.ds(r0 + nfull * ck, rem)],
                        rows_v[0].at[pl.ds(0, rem)])
        pltpu.sync_copy(rows_v[0].at[pl.ds(0, rem)],
                        out_hbm.at[c, pl.ds(r0 + nfull * ck, rem)])


def _agg_call(y2, src2, dst2, w2, *, hc):
    ck = 128 if hc == 32 else 256
    cb = ck // 128
    body = functools.partial(_agg_body, hc=hc, ck=ck)
    return pl.kernel(
        body,
        out_type=jax.ShapeDtypeStruct((NC, NPAD, hc), jnp.float32),
        mesh=_mesh(),
        scratch_types=[
            pltpu.VMEM_SHARED((NPAD, hc), jnp.float32),
            [pltpu.VMEM((cb, 128), jnp.int32) for _ in range(4)],
            [pltpu.VMEM((cb, 128), jnp.int32) for _ in range(4)],
            [pltpu.VMEM((cb, 128), jnp.int32) for _ in range(4)],
            [pltpu.VMEM((cb, 128), jnp.float32) for _ in range(4)],
            [pltpu.VMEM((ck, hc), jnp.float32) for _ in range(4)],
            [pltpu.SemaphoreType.DMA for _ in range(4)],
            [pltpu.SemaphoreType.DMA for _ in range(4)],
            [pltpu.SemaphoreType.DMA for _ in range(4)],
        ],
        compiler_params=_SC_PARAMS,
        name=f"sc_agg{hc}",
    )(y2, src2, dst2, w2)


def _head_body(emb_ref, d0_ref, d1_ref, t_ref, dis_ref):
    deg = d0_ref[...] + d1_ref[...] + 1.0
    dis = jnp.where(deg > 0, lax.rsqrt(deg), 0.0)
    dis_ref[...] = dis
    t_ref[...] = dis * emb_ref[...]


def _head_call(emb_p, deg0, deg1):
    return pl.pallas_call(
        _head_body,
        grid=(GRID,),
        in_specs=[
            pl.BlockSpec((BN, EMB), lambda i: (i, 0)),
            pl.BlockSpec((BN, 1), lambda i: (i, 0)),
            pl.BlockSpec((BN, 1), lambda i: (i, 0)),
        ],
        out_specs=[
            pl.BlockSpec((BN, EMB), lambda i: (i, 0)),
            pl.BlockSpec((BN, 1), lambda i: (i, 0)),
        ],
        out_shape=[
            jax.ShapeDtypeStruct((NPAD, EMB), jnp.float32),
            jax.ShapeDtypeStruct((NPAD, 1), jnp.float32),
        ],
        name="tc_head",
    )(emb_p, deg0, deg1)


def _enc_body(sl_ref, sr_ref, t_ref, dis_ref, w1_ref, b_ref, w2_ref, out_ref):
    dis = dis_ref[...]
    u = jnp.concatenate([sl_ref[...], sr_ref[...]], axis=1) + t_ref[...]
    x = jnp.maximum(dis * jnp.dot(u, w1_ref[...],
                                  preferred_element_type=jnp.float32)
                    + b_ref[...], 0.0)
    out_ref[...] = dis * jnp.dot(x, w2_ref[...],
                                 preferred_element_type=jnp.float32)


def _enc_call(sL, sR, t, dis, W1, b, W2):
    hcin = sL.shape[-1]
    din = 2 * hcin
    dmid = W1.shape[1]
    dout = W2.shape[1]
    return pl.pallas_call(
        _enc_body,
        grid=(GRID,),
        in_specs=[
            pl.BlockSpec((BN, hcin), lambda i: (i, 0)),
            pl.BlockSpec((BN, hcin), lambda i: (i, 0)),
            pl.BlockSpec((BN, din), lambda i: (i, 0)),
            pl.BlockSpec((BN, 1), lambda i: (i, 0)),
            pl.BlockSpec((din, dmid), lambda i: (0, 0)),
            pl.BlockSpec((1, dmid), lambda i: (0, 0)),
            pl.BlockSpec((dmid, dout), lambda i: (0, 0)),
        ],
        out_specs=pl.BlockSpec((BN, dout), lambda i: (i, 0)),
        out_shape=jax.ShapeDtypeStruct((NPAD, dout), jnp.float32),
        name=f"tc_enc_{din}_{dmid}_{dout}",
    )(sL, sR, t, dis, W1, b, W2)


def _zt_body(sl_ref, sr_ref, y_ref, dis_ref, b_ref, z_ref, t_ref):
    dis = dis_ref[...]
    agg = jnp.concatenate([sl_ref[...], sr_ref[...]], axis=1) + y_ref[...]
    z = jnp.maximum(dis * agg + b_ref[...], 0.0)
    z_ref[...] = z
    t_ref[...] = dis * z


def _zt_call(sL, sR, y, dis, b):
    hcin = sL.shape[-1]
    din = 2 * hcin
    return pl.pallas_call(
        _zt_body,
        grid=(GRID,),
        in_specs=[
            pl.BlockSpec((BN, hcin), lambda i: (i, 0)),
            pl.BlockSpec((BN, hcin), lambda i: (i, 0)),
            pl.BlockSpec((BN, din), lambda i: (i, 0)),
            pl.BlockSpec((BN, 1), lambda i: (i, 0)),
            pl.BlockSpec((1, din), lambda i: (0, 0)),
        ],
        out_specs=[
            pl.BlockSpec((BN, din), lambda i: (i, 0)),
            pl.BlockSpec((BN, din), lambda i: (i, 0)),
        ],
        out_shape=[
            jax.ShapeDtypeStruct((NPAD, din), jnp.float32),
            jax.ShapeDtypeStruct((NPAD, din), jnp.float32),
        ],
        name="tc_zt",
    )(sL, sR, y, dis, b)


def _fin_body(sl_ref, sr_ref, y_ref, dis_ref, b_ref, out_ref):
    agg = jnp.concatenate([sl_ref[...], sr_ref[...]], axis=1) + y_ref[...]
    out_ref[...] = dis_ref[...] * agg + b_ref[...]


def _fin_call(sL, sR, y, dis, b):
    hcin = sL.shape[-1]
    din = 2 * hcin
    return pl.pallas_call(
        _fin_body,
        grid=(GRID,),
        in_specs=[
            pl.BlockSpec((BN, hcin), lambda i: (i, 0)),
            pl.BlockSpec((BN, hcin), lambda i: (i, 0)),
            pl.BlockSpec((BN, din), lambda i: (i, 0)),
            pl.BlockSpec((BN, 1), lambda i: (i, 0)),
            pl.BlockSpec((1, din), lambda i: (0, 0)),
        ],
        out_specs=pl.BlockSpec((BN, din), lambda i: (i, 0)),
        out_shape=jax.ShapeDtypeStruct((NPAD, din), jnp.float32),
        name="tc_final",
    )(sL, sR, y, dis, b)


def kernel(edge_index, edge_weight, embedding, W1e, b1e, W2e, b2e,
           W1d, b1d, W2d, b2d):
    src = edge_index[0]
    dst = edge_index[1]
    pad = EPAD + 16 * 128 - E
    src2 = jnp.concatenate([src, jnp.zeros((pad,), src.dtype)]).reshape(
        EDGE_ROWS + 16, 128)
    dst2 = jnp.concatenate([dst, jnp.zeros((pad,), dst.dtype)]).reshape(
        EDGE_ROWS + 16, 128)
    w2 = jnp.concatenate([edge_weight,
                          jnp.zeros((pad,), edge_weight.dtype)]).reshape(
        EDGE_ROWS + 16, 128)
    emb_p = jnp.pad(embedding, ((0, NPAD - N), (0, 0)))

    deg2 = _deg_call(dst2, w2).reshape(NC, NPAD)
    deg0 = deg2[0].reshape(NPAD, 1)
    deg1 = deg2[1].reshape(NPAD, 1)

    t1, dis = _head_call(emb_p, deg0, deg1)
    S1 = _agg_call(t1.reshape(2 * NPAD, EMB // 2), src2, dst2, w2, hc=EMB // 2)
    y2 = _enc_call(S1[0], S1[1], t1, dis, W1e, b1e.reshape(1, HID), W2e)

    S2 = _agg_call(y2.reshape(2 * NPAD, EMB // 2), src2, dst2, w2, hc=EMB // 2)
    z, t3 = _zt_call(S2[0], S2[1], y2, dis, b2e.reshape(1, EMB))

    S3 = _agg_call(t3.reshape(2 * NPAD, EMB // 2), src2, dst2, w2, hc=EMB // 2)
    y4 = _enc_call(S3[0], S3[1], t3, dis, W1d, b1d.reshape(1, HID), W2d)

    S4 = _agg_call(y4.reshape(2 * NPAD, IN_DIM // 2), src2, dst2, w2,
                   hc=IN_DIM // 2)
    recon = _fin_call(S4[0], S4[1], y4, dis, b2d.reshape(1, IN_DIM))
    return recon[:N], z[:N]
